# Initial kernel scaffold; baseline (speedup 1.0000x reference)
import jax
import jax.numpy as jnp
from jax.experimental import pallas as pl


def kernel(x, edge_index, W1a, b1a, g1, be1, W1b, b1b, W2a, b2a, g2, be2, W2b, b2b, W3a, b3a, g3, be3, W3b, b3b, Wo, bo):
    raise NotImplementedError("write your pallas kernel here")


# speedup vs baseline: 11.9119x; 11.9119x over previous
import functools

import jax
import jax.numpy as jnp
from jax import lax
from jax.experimental import pallas as pl
from jax.experimental.pallas import tpu as pltpu
from jax.experimental.pallas import tpu_sc as plsc

N = 10000
D = 128
OUT = 2
NC = 2
NS = 16
NW = NC * NS
E = 320000
CHUNK = 64
NCH = 159
KPG = CHUNK // 16
EPW = NCH * CHUNK
EPAD = NW * EPW - E
NPAD = 10112
RPT = NPAD // NS
NBUF = 3


def _sc_agg_body(h_hbm, edge_hbm, out_hbm, pvm, si, di,
                 rb0, rb1, rb2, acc, g0, g1, g2):
    c = lax.axis_index("c")
    s = lax.axis_index("s")
    wid = c * NS + s
    r0 = s * RPT
    pltpu.sync_copy(edge_hbm.at[wid], pvm)
    pltpu.sync_copy(h_hbm.at[pl.ds(r0, RPT)], acc.at[pl.ds(r0, RPT)])
    plsc.subcore_barrier()

    rbs = [rb0, rb1, rb2]
    gs = [g0, g1, g2]

    def unpack(j, b):
        for t in range(KPG):
            v = pvm[j, pl.ds(t * 16, 16)]
            si[b, pl.ds(t * 16, 16)] = v & 0x3FFF
            di[b, pl.ds(t * 16, 16)] = lax.shift_right_logical(v, 14)

    for b in range(NBUF):
        unpack(b, b)
        pltpu.async_copy(h_hbm.at[si.at[b]], rbs[b], gs[b])

    def body(i, carry):
        j0 = NBUF * i
        for b in range(NBUF):
            pltpu.make_async_copy(h_hbm.at[si.at[b]], rbs[b], gs[b]).wait()
            pltpu.sync_copy(rbs[b], acc.at[di.at[b]], add=True)
            unpack(j0 + b + NBUF, b)
            pltpu.async_copy(h_hbm.at[si.at[b]], rbs[b], gs[b])
        return carry

    lax.fori_loop(0, NCH // NBUF - 1, body, 0)
    for b in range(NBUF):
        pltpu.make_async_copy(h_hbm.at[si.at[b]], rbs[b], gs[b]).wait()
        pltpu.sync_copy(rbs[b], acc.at[di.at[b]], add=True)

    plsc.subcore_barrier()
    pltpu.sync_copy(acc.at[pl.ds(r0, RPT)], out_hbm.at[c, pl.ds(r0, RPT)])


@functools.cache
def _make_sc_agg():
    return pl.kernel(
        _sc_agg_body,
        out_type=jax.ShapeDtypeStruct((NC, NPAD, D), jnp.float32),
        mesh=plsc.VectorSubcoreMesh(core_axis_name="c", subcore_axis_name="s",
                                    num_cores=NC, num_subcores=NS),
        scratch_types=(
            [pltpu.VMEM((NCH, CHUNK), jnp.int32)]
            + [pltpu.VMEM((NBUF, CHUNK), jnp.int32)]
            + [pltpu.VMEM((NBUF, CHUNK), jnp.int32)]
            + [pltpu.VMEM((CHUNK, D), jnp.float32) for _ in range(NBUF)]
            + [pltpu.VMEM_SHARED((NPAD, D), jnp.float32)]
            + [pltpu.SemaphoreType.DMA for _ in range(NBUF)]
        ),
    )


def _sc_agg(h, edges_r):
    return _make_sc_agg()(h, edges_r)


def _mlp_head(h_ref, p_ref, wa_ref, ba_ref, g_ref, be_ref):
    mask = lax.broadcasted_iota(jnp.int32, (NPAD, 1), 0) < N
    z = p_ref[0] + p_ref[1] - h_ref[...]
    z = jnp.where(mask, z, 0.0)
    a = jnp.dot(z, wa_ref[...], preferred_element_type=jnp.float32) + ba_ref[...]
    am = jnp.where(mask, a, 0.0)
    mean = jnp.sum(am, axis=0, keepdims=True) / N
    cent = a - mean
    var = jnp.sum(jnp.where(mask, cent * cent, 0.0), axis=0, keepdims=True) / N
    an = cent * lax.rsqrt(var + 1e-5) * g_ref[...] + be_ref[...]
    return jnp.maximum(an, 0.0), mask


def _dense_block(h_ref, p_ref, wa_ref, ba_ref, g_ref, be_ref, wb_ref, bb_ref, o_ref):
    an, mask = _mlp_head(h_ref, p_ref, wa_ref, ba_ref, g_ref, be_ref)
    h2 = jnp.dot(an, wb_ref[...], preferred_element_type=jnp.float32) + bb_ref[...]
    o_ref[...] = jnp.where(mask, jnp.maximum(h2, 0.0), 0.0)


def _final_block(h_ref, p_ref, wa_ref, ba_ref, g_ref, be_ref, wb_ref, bb_ref,
                 wo_ref, bo_ref, o_ref):
    an, _ = _mlp_head(h_ref, p_ref, wa_ref, ba_ref, g_ref, be_ref)
    wf = jnp.dot(wb_ref[...], wo_ref[...], preferred_element_type=jnp.float32)
    bf = jnp.dot(bb_ref[...], wo_ref[...], preferred_element_type=jnp.float32) + bo_ref[...]
    o_ref[...] = jnp.dot(an[:N, :], wf, preferred_element_type=jnp.float32) + bf


_dense = pl.pallas_call(
    _dense_block,
    out_shape=jax.ShapeDtypeStruct((NPAD, D), jnp.float32),
)

_final = pl.pallas_call(
    _final_block,
    out_shape=jax.ShapeDtypeStruct((N, OUT), jnp.float32),
)


def kernel(x, edge_index, W1a, b1a, g1, be1, W1b, b1b, W2a, b2a, g2, be2,
           W2b, b2b, W3a, b3a, g3, be3, W3b, b3b, Wo, bo):
    src = edge_index[0]
    dst = edge_index[1]
    pad_ar = jnp.arange(EPAD, dtype=jnp.int32)
    src_p = jnp.concatenate([src, (pad_ar * 131) % N])
    dst_p = jnp.concatenate([dst, N + (pad_ar % NW)])
    edges_r = ((dst_p << 14) | src_p).reshape(NW, NCH, CHUNK)

    hpad = jnp.zeros((NPAD, D), jnp.float32).at[:N].set(x)
    b1a, b1b = b1a.reshape(1, D), b1b.reshape(1, D)
    b2a, b2b = b2a.reshape(1, D), b2b.reshape(1, D)
    b3a, b3b = b3a.reshape(1, D), b3b.reshape(1, D)
    g1, be1 = g1.reshape(1, D), be1.reshape(1, D)
    g2, be2 = g2.reshape(1, D), be2.reshape(1, D)
    g3, be3 = g3.reshape(1, D), be3.reshape(1, D)
    bo = bo.reshape(1, OUT)

    p = _sc_agg(hpad, edges_r)
    h = _dense(hpad, p, W1a, b1a, g1, be1, W1b, b1b)
    p = _sc_agg(h, edges_r)
    h = _dense(h, p, W2a, b2a, g2, be2, W2b, b2b)
    p = _sc_agg(h, edges_r)
    return _final(h, p, W3a, b3a, g3, be3, W3b, b3b, Wo, bo)


# speedup vs baseline: 12.3423x; 1.0361x over previous
import functools

import jax
import jax.numpy as jnp
from jax import lax
from jax.experimental import pallas as pl
from jax.experimental.pallas import tpu as pltpu
from jax.experimental.pallas import tpu_sc as plsc

N = 10000
D = 128
OUT = 2
NC = 2
NS = 16
NW = NC * NS
E = 320000
CHUNK = 80
NCH = 128
KPG = CHUNK // 16
EPW = NCH * CHUNK
EPAD = NW * EPW - E
NPAD = 10112
RPT = NPAD // NS
NBUF = 3


def _sc_agg_body(h_hbm, edge_hbm, out_hbm, pvm, si, di,
                 rb0, rb1, rb2, acc, g0, g1, g2):
    c = lax.axis_index("c")
    s = lax.axis_index("s")
    wid = c * NS + s
    r0 = s * RPT
    pltpu.sync_copy(edge_hbm.at[wid], pvm)
    pltpu.sync_copy(h_hbm.at[pl.ds(r0, RPT)], acc.at[pl.ds(r0, RPT)])
    plsc.subcore_barrier()

    rbs = [rb0, rb1, rb2]
    gs = [g0, g1, g2]

    def unpack(j, b):
        for t in range(KPG):
            v = pvm[j, pl.ds(t * 16, 16)]
            si[b, pl.ds(t * 16, 16)] = v & 0x3FFF
            di[b, pl.ds(t * 16, 16)] = lax.shift_right_logical(v, 14)

    for b in range(NBUF):
        unpack(b, b)
        pltpu.async_copy(h_hbm.at[si.at[b]], rbs[b], gs[b])

    def body(i, carry):
        j0 = NBUF * i
        for b in range(NBUF):
            pltpu.make_async_copy(h_hbm.at[si.at[b]], rbs[b], gs[b]).wait()
            pltpu.sync_copy(rbs[b], acc.at[di.at[b]], add=True)
            unpack(j0 + b + NBUF, b)
            pltpu.async_copy(h_hbm.at[si.at[b]], rbs[b], gs[b])
        return carry

    lax.fori_loop(0, NCH // NBUF - 1, body, 0)
    for b in range(NBUF):
        pltpu.make_async_copy(h_hbm.at[si.at[b]], rbs[b], gs[b]).wait()
        pltpu.sync_copy(rbs[b], acc.at[di.at[b]], add=True)

    plsc.subcore_barrier()
    pltpu.sync_copy(acc.at[pl.ds(r0, RPT)], out_hbm.at[c, pl.ds(r0, RPT)])


@functools.cache
def _make_sc_agg():
    return pl.kernel(
        _sc_agg_body,
        out_type=jax.ShapeDtypeStruct((NC, NPAD, D), jnp.float32),
        mesh=plsc.VectorSubcoreMesh(core_axis_name="c", subcore_axis_name="s",
                                    num_cores=NC, num_subcores=NS),
        scratch_types=(
            [pltpu.VMEM((NCH, CHUNK), jnp.int32)]
            + [pltpu.VMEM((NBUF, CHUNK), jnp.int32)]
            + [pltpu.VMEM((NBUF, CHUNK), jnp.int32)]
            + [pltpu.VMEM((CHUNK, D), jnp.float32) for _ in range(NBUF)]
            + [pltpu.VMEM_SHARED((NPAD, D), jnp.float32)]
            + [pltpu.SemaphoreType.DMA for _ in range(NBUF)]
        ),
    )


def _sc_agg(h, edges_r):
    return _make_sc_agg()(h, edges_r)


def _mlp_head(h_ref, p_ref, wa_ref, ba_ref, g_ref, be_ref):
    mask = lax.broadcasted_iota(jnp.int32, (NPAD, 1), 0) < N
    z = p_ref[0] + p_ref[1] - h_ref[...]
    z = jnp.where(mask, z, 0.0)
    a = jnp.dot(z, wa_ref[...], preferred_element_type=jnp.float32) + ba_ref[...]
    am = jnp.where(mask, a, 0.0)
    mean = jnp.sum(am, axis=0, keepdims=True) / N
    cent = a - mean
    var = jnp.sum(jnp.where(mask, cent * cent, 0.0), axis=0, keepdims=True) / N
    an = cent * lax.rsqrt(var + 1e-5) * g_ref[...] + be_ref[...]
    return jnp.maximum(an, 0.0), mask


def _dense_block(h_ref, p_ref, wa_ref, ba_ref, g_ref, be_ref, wb_ref, bb_ref, o_ref):
    an, mask = _mlp_head(h_ref, p_ref, wa_ref, ba_ref, g_ref, be_ref)
    h2 = jnp.dot(an, wb_ref[...], preferred_element_type=jnp.float32) + bb_ref[...]
    o_ref[...] = jnp.where(mask, jnp.maximum(h2, 0.0), 0.0)


def _final_block(h_ref, p_ref, wa_ref, ba_ref, g_ref, be_ref, wb_ref, bb_ref,
                 wo_ref, bo_ref, o_ref):
    an, _ = _mlp_head(h_ref, p_ref, wa_ref, ba_ref, g_ref, be_ref)
    wf = jnp.dot(wb_ref[...], wo_ref[...], preferred_element_type=jnp.float32)
    bf = jnp.dot(bb_ref[...], wo_ref[...], preferred_element_type=jnp.float32) + bo_ref[...]
    o_ref[...] = jnp.dot(an[:N, :], wf, preferred_element_type=jnp.float32) + bf


_dense = pl.pallas_call(
    _dense_block,
    out_shape=jax.ShapeDtypeStruct((NPAD, D), jnp.float32),
)

_final = pl.pallas_call(
    _final_block,
    out_shape=jax.ShapeDtypeStruct((N, OUT), jnp.float32),
)


def kernel(x, edge_index, W1a, b1a, g1, be1, W1b, b1b, W2a, b2a, g2, be2,
           W2b, b2b, W3a, b3a, g3, be3, W3b, b3b, Wo, bo):
    src = edge_index[0]
    dst = edge_index[1]
    pad_ar = jnp.arange(EPAD, dtype=jnp.int32)
    src_p = jnp.concatenate([src, (pad_ar * 131) % N])
    dst_p = jnp.concatenate([dst, N + (pad_ar % NW)])
    edges_r = ((dst_p << 14) | src_p).reshape(NW, NCH, CHUNK)

    hpad = jnp.zeros((NPAD, D), jnp.float32).at[:N].set(x)
    b1a, b1b = b1a.reshape(1, D), b1b.reshape(1, D)
    b2a, b2b = b2a.reshape(1, D), b2b.reshape(1, D)
    b3a, b3b = b3a.reshape(1, D), b3b.reshape(1, D)
    g1, be1 = g1.reshape(1, D), be1.reshape(1, D)
    g2, be2 = g2.reshape(1, D), be2.reshape(1, D)
    g3, be3 = g3.reshape(1, D), be3.reshape(1, D)
    bo = bo.reshape(1, OUT)

    p = _sc_agg(hpad, edges_r)
    h = _dense(hpad, p, W1a, b1a, g1, be1, W1b, b1b)
    p = _sc_agg(h, edges_r)
    h = _dense(h, p, W2a, b2a, g2, be2, W2b, b2b)
    p = _sc_agg(h, edges_r)
    return _final(h, p, W3a, b3a, g3, be3, W3b, b3b, Wo, bo)
